# baseline (speedup 1.0000x reference)
import functools

import jax
import jax.numpy as jnp
from jax import lax
from jax.experimental import pallas as pl
from jax.experimental.pallas import tpu as pltpu
from jax.experimental.pallas import tpu_sc as plsc

NUM_TOP = 300
OUT_PAD = 304
OCT = 640
CAP_OCT = 384
CAP_C = 512
LANES = 16

_I32_FLIP = 0x7FFFFFFF


def _sum_body(x_ref, om_ref, lin_ref):
    x = x_ref[0]
    qb, c = x.shape
    x3 = x.reshape(qb // 8, 8, c)
    om_ref[0, 0, 0, :] = jnp.max(x3, axis=(1, 2))
    rows = [
        jnp.concatenate([x3[:, 0, :], x3[:, 1, 0:48]], axis=-1),
        jnp.concatenate([x3[:, 1, 48:80], x3[:, 2, :], x3[:, 3, 0:16]], axis=-1),
        jnp.concatenate([x3[:, 3, 16:80], x3[:, 4, 0:64]], axis=-1),
        jnp.concatenate([x3[:, 4, 64:80], x3[:, 5, :], x3[:, 6, 0:32]], axis=-1),
        jnp.concatenate([x3[:, 6, 32:80], x3[:, 7, :]], axis=-1),
    ]
    lin_ref[0, 0] = jnp.stack(rows, axis=1)


def _thresh_body(rm_ref, t_ref, *, nb):
    rm = rm_ref[...]
    u = lax.bitcast_convert_type(rm, jnp.int32)
    key = jnp.where(u >= 0, u, u ^ jnp.int32(_I32_FLIP))

    def it(_, lohi):
        lo, hi = lohi
        fl = (lo >> 1) + (hi >> 1) + (lo & hi & 1)
        mid = fl + ((lo ^ hi) & 1)
        cnt = jnp.sum((key >= mid).astype(jnp.int32), axis=(1, 2, 3), keepdims=True)
        ok = cnt >= NUM_TOP
        return jnp.where(ok, mid, lo), jnp.where(ok, hi, mid - 1)

    lo0 = jnp.full((nb, 1, 1, 1), jnp.iinfo(jnp.int32).min, jnp.int32)
    hi0 = jnp.full((nb, 1, 1, 1), jnp.iinfo(jnp.int32).max, jnp.int32)
    k_fin, _ = lax.fori_loop(0, 32, it, (lo0, hi0))
    ub = jnp.where(k_fin >= 0, k_fin, k_fin ^ jnp.int32(_I32_FLIP))
    t = lax.bitcast_convert_type(ub, jnp.float32)
    t_ref[...] = jnp.broadcast_to(t.reshape(nb, 1), (nb, LANES))


def _iota16():
    return lax.iota(jnp.int32, LANES)


def _bcast(x):
    return jnp.full((LANES,), x, jnp.int32)


def _sc_body(octmax_hbm, thresh_hbm, lin_hbm, boxes_hbm, scale_hbm,
             lab_out, box_out, sc_out,
             rm_v, t_v, scale_v, cand_v, cand8_v, rows_a, rows_b, cv_v, ci_v,
             sv_v, si_v, lab_v, sc_v, qg_v, bl_v, bx_v, bxo_v, sem_a, sem_b,
             *, nb, q, c, gp):
    wid = lax.axis_index("s") * 2 + lax.axis_index("c")
    g_per_b = q * c // OCT
    n_r8 = CAP_OCT * 5
    n_ch = n_r8 // 128

    @pl.when(wid < nb)
    def _():
        b = wid
        pltpu.sync_copy(octmax_hbm.at[b], rm_v)
        pltpu.sync_copy(thresh_hbm.at[b], t_v)
        pltpu.sync_copy(scale_hbm.at[b], scale_v)
        t_vec = t_v[...]
        iota = _iota16()
        oct0 = b * g_per_b
        row0 = b * q

        for i in range(CAP_OCT // LANES):
            cand_v[pl.ds(i * LANES, LANES)] = _bcast(oct0)
        neg_inf = jnp.full((LANES,), -jnp.inf, jnp.float32)
        for i in range(CAP_C // LANES):
            cv_v[pl.ds(i * LANES, LANES)] = neg_inf
            ci_v[pl.ds(i * LANES, LANES)] = _bcast(0)

        def coct(i, off):
            v = rm_v[pl.ds(i * LANES, LANES)]
            m = v >= t_vec
            mi = m.astype(jnp.int32)
            pos = plsc.cumsum(mi) - mi
            dst = jnp.minimum(off + pos, CAP_OCT - 1)
            plsc.store_scatter(cand_v, [dst], oct0 + i * LANES + iota, mask=m)
            return off + plsc.all_reduce_population_count(m)[0]

        n_oct = lax.fori_loop(0, gp // LANES, coct, jnp.int32(0))

        lrow0 = b * (q * c // 128)

        def expand(t, _):
            j = t * LANES + iota
            j5 = j // 5
            o = plsc.load_gather(cand_v, [j5])
            r = (o - oct0) * 5 + lrow0 + (j - j5 * 5)
            pad_r = lrow0 + (j & 1023)
            cand8_v[pl.ds(t * LANES, LANES)] = jnp.where(j < n_oct * 5, r, pad_r)
            return 0

        lax.fori_loop(0, n_r8 // LANES, expand, 0)

        bufs = (rows_a, rows_b)
        sems = (sem_a, sem_b)

        def fire(ch):
            return pltpu.async_copy(
                lin_hbm.at[cand8_v.at[pl.ds(ch * 128, 128)]],
                bufs[ch % 2], sems[ch % 2])

        cps = {0: fire(0)}
        off2 = jnp.int32(0)
        for ch in range(n_ch):
            if ch + 1 < n_ch:
                cps[ch + 1] = fire(ch + 1)
            cps[ch].wait()
            buf = bufs[ch % 2]

            def ext(j, o2, _ch=ch, _buf=buf):
                r = plsc.load_gather(cand8_v, [_bcast(_ch * 128 + j)])
                base = (r - lrow0) * 128
                vs, ms, cnts, poss = [], [], [], []
                for k in range(128 // LANES):
                    v = plsc.load_gather(_buf, [_bcast(j), k * LANES + iota])
                    m = v >= t_vec
                    mi = m.astype(jnp.int32)
                    vs.append(v)
                    ms.append(m)
                    cnts.append(plsc.all_reduce_population_count(m)[0])
                    poss.append(plsc.cumsum(mi) - mi)
                off = o2
                for k in range(128 // LANES):
                    dst = jnp.minimum(off + poss[k], CAP_C - 1)
                    plsc.store_scatter(cv_v, [dst], vs[k], mask=ms[k])
                    plsc.store_scatter(ci_v, [dst], base + k * LANES + iota, mask=ms[k])
                    off = off + cnts[k]
                return off

            nj = jnp.clip(n_oct * 5 - ch * 128, 0, 128)
            off2 = lax.fori_loop(0, nj, ext, off2)
        n_c = off2

        n_cv = (n_c + LANES - 1) // LANES

        def rank_iv(iv, _):
            sl = iv * LANES + iota
            vi = plsc.load_gather(cv_v, [sl])
            xi = plsc.load_gather(ci_v, [sl])

            def inner(jv, acc):
                jsl = jv * LANES + iota
                vj = plsc.load_gather(cv_v, [jsl])
                xj = plsc.load_gather(ci_v, [jsl])
                dn = lax.GatherDimensionNumbers(
                    offset_dims=(), collapsed_slice_dims=(0,), start_index_map=(0,))
                for l in range(LANES):
                    lane = _bcast(l)[:, None]
                    vjb = lax.gather(vj, lane, dn, (1,),
                                     mode=lax.GatherScatterMode.PROMISE_IN_BOUNDS)
                    xjb = lax.gather(xj, lane, dn, (1,),
                                     mode=lax.GatherScatterMode.PROMISE_IN_BOUNDS)
                    beat = (vjb > vi) | ((vjb == vi) & (xjb < xi))
                    acc = acc + beat.astype(jnp.int32)
                return acc

            rank = lax.fori_loop(0, n_cv, inner, jnp.zeros((LANES,), jnp.int32))
            m = (rank < OUT_PAD) & (sl < n_c)
            plsc.store_scatter(sv_v, [rank], vi, mask=m)
            plsc.store_scatter(si_v, [rank], xi, mask=m)
            return 0

        lax.fori_loop(0, n_cv, rank_iv, 0)

        for s in range(OUT_PAD // LANES):
            sl = pl.ds(s * LANES, LANES)
            v = sv_v[sl]
            idx = si_v[sl]
            lab_v[sl] = idx - (idx // c) * c
            sc_v[sl] = 1.0 / (1.0 + jnp.exp(-v))
            qq = jnp.clip(idx // c, 0, q - 1)
            bflat = (row0 + qq) * 4
            qg_v[sl] = bflat >> 7
            bl_v[sl] = bflat & 127
        for s in range(OUT_PAD // LANES, 384 // LANES):
            qg_v[pl.ds(s * LANES, LANES)] = _bcast((row0 * 4) >> 7)

        copies = []
        for ch in range(384 // 128):
            copies.append(pltpu.async_copy(
                boxes_hbm.at[qg_v.at[pl.ds(ch * 128, 128)]],
                bx_v.at[pl.ds(ch * 128, 128)], sem_a))
        for cp in copies:
            cp.wait()

        box_of_lane = iota >> 2
        par = iota & 1
        sign = jnp.where((iota & 3) < 2, jnp.float32(-0.5), jnp.float32(0.5))
        sc_scale = scale_v[...]
        for g in range(OUT_PAD // 4):
            slot = 4 * g + box_of_lane
            blane = plsc.load_gather(bl_v, [slot])
            ctr = plsc.load_gather(bx_v, [slot, blane + par])
            ext2 = plsc.load_gather(bx_v, [slot, blane + 2 + par])
            bxo_v[pl.ds(g * LANES, LANES)] = (ctr + sign * ext2) * sc_scale

        pltpu.sync_copy(lab_v, lab_out.at[b])
        pltpu.sync_copy(sc_v, sc_out.at[b])
        pltpu.sync_copy(bxo_v, box_out.at[b])


def kernel(pred_logits, pred_boxes, orig_target_sizes):
    nb, q, c = pred_logits.shape
    qb = 4000
    nj = q // qb
    g_per_b = q * c // OCT
    gp = 2560

    octmax8, lin8 = pl.pallas_call(
        _sum_body,
        grid=(nb, nj),
        in_specs=[pl.BlockSpec((1, qb, c), lambda b, j: (b, j, 0))],
        out_specs=[
            pl.BlockSpec((1, 1, 1, qb // 8), lambda b, j: (b, j, 0, 0)),
            pl.BlockSpec((1, 1, qb // 8, 5, 128), lambda b, j: (b, j, 0, 0, 0)),
        ],
        out_shape=[
            jax.ShapeDtypeStruct((nb, nj, 1, qb // 8), jnp.float32),
            jax.ShapeDtypeStruct((nb, nj, qb // 8, 5, 128), jnp.float32),
        ],
    )(pred_logits)

    thresh = pl.pallas_call(
        functools.partial(_thresh_body, nb=nb),
        out_shape=jax.ShapeDtypeStruct((nb, LANES), jnp.float32),
    )(octmax8)

    octmax = jnp.pad(octmax8.reshape(nb, g_per_b), ((0, 0), (0, gp - g_per_b)),
                     constant_values=-jnp.inf)
    lin = lin8.reshape(nb * q * c // 128, 128)
    boxes128 = pred_boxes.reshape(nb * q * 4 // 128, 128)
    scale16 = jnp.tile(orig_target_sizes.astype(jnp.float32), (1, 8))

    mesh = plsc.VectorSubcoreMesh(core_axis_name="c", subcore_axis_name="s")
    sc = functools.partial(
        pl.kernel,
        out_type=[
            jax.ShapeDtypeStruct((nb, OUT_PAD), jnp.int32),
            jax.ShapeDtypeStruct((nb, OUT_PAD * 4), jnp.float32),
            jax.ShapeDtypeStruct((nb, OUT_PAD), jnp.float32),
        ],
        mesh=mesh,
        compiler_params=pltpu.CompilerParams(needs_layout_passes=False, use_tc_tiling_on_sc=True),
        scratch_types=[
            pltpu.VMEM((gp,), jnp.float32),
            pltpu.VMEM((LANES,), jnp.float32),
            pltpu.VMEM((LANES,), jnp.float32),
            pltpu.VMEM((CAP_OCT,), jnp.int32),
            pltpu.VMEM((CAP_OCT * 5,), jnp.int32),
            pltpu.VMEM((128, 128), jnp.float32),
            pltpu.VMEM((128, 128), jnp.float32),
            pltpu.VMEM((CAP_C,), jnp.float32),
            pltpu.VMEM((CAP_C,), jnp.int32),
            pltpu.VMEM((OUT_PAD,), jnp.float32),
            pltpu.VMEM((OUT_PAD,), jnp.int32),
            pltpu.VMEM((OUT_PAD,), jnp.int32),
            pltpu.VMEM((OUT_PAD,), jnp.float32),
            pltpu.VMEM((384,), jnp.int32),
            pltpu.VMEM((OUT_PAD,), jnp.int32),
            pltpu.VMEM((384, 128), jnp.float32),
            pltpu.VMEM((OUT_PAD * 4,), jnp.float32),
            pltpu.SemaphoreType.DMA,
            pltpu.SemaphoreType.DMA,
        ],
    )(functools.partial(_sc_body, nb=nb, q=q, c=c, gp=gp))

    labels_p, boxes_p, scores_p = sc(octmax, thresh, lin, boxes128, scale16)
    return (labels_p[:, :NUM_TOP],
            boxes_p.reshape(nb, OUT_PAD, 4)[:, :NUM_TOP],
            scores_p[:, :NUM_TOP])


# speedup vs baseline: 1.2518x; 1.2518x over previous
import functools

import jax
import jax.numpy as jnp
from jax import lax
from jax.experimental import pallas as pl
from jax.experimental.pallas import tpu as pltpu
from jax.experimental.pallas import tpu_sc as plsc

NUM_TOP = 300
OUT_PAD = 304
OCT = 640
CAP_OCT = 384
CAP_C = 512
LANES = 16

_I32_FLIP = 0x7FFFFFFF


def _sum_body(x_ref, om_ref, lin_ref):
    x = x_ref[0]
    qb, c = x.shape
    om_ref[0, 0, 0, :] = jnp.max(x.reshape(qb // 8, 8, c), axis=(1, 2))
    pad = jnp.full((qb, 128 - c), -jnp.inf, jnp.float32)
    lin_ref[0, 0] = jnp.concatenate([x, pad], axis=-1)


def _thresh_body(rm_ref, t_ref, *, nb):
    rm = rm_ref[...]
    u = lax.bitcast_convert_type(rm, jnp.int32)
    key = jnp.where(u >= 0, u, u ^ jnp.int32(_I32_FLIP))

    def it(_, lohi):
        lo, hi = lohi
        fl = (lo >> 1) + (hi >> 1) + (lo & hi & 1)
        mid = fl + ((lo ^ hi) & 1)
        cnt = jnp.sum((key >= mid).astype(jnp.int32), axis=(1, 2, 3), keepdims=True)
        ok = cnt >= NUM_TOP
        return jnp.where(ok, mid, lo), jnp.where(ok, hi, mid - 1)

    lo0 = jnp.full((nb, 1, 1, 1), jnp.iinfo(jnp.int32).min, jnp.int32)
    hi0 = jnp.full((nb, 1, 1, 1), jnp.iinfo(jnp.int32).max, jnp.int32)
    k_fin, _ = lax.fori_loop(0, 32, it, (lo0, hi0))
    ub = jnp.where(k_fin >= 0, k_fin, k_fin ^ jnp.int32(_I32_FLIP))
    t = lax.bitcast_convert_type(ub, jnp.float32)
    t_ref[...] = jnp.broadcast_to(t.reshape(nb, 1), (nb, LANES))


def _iota16():
    return lax.iota(jnp.int32, LANES)


def _bcast(x):
    return jnp.full((LANES,), x, jnp.int32)


def _sc_body(octmax_hbm, thresh_hbm, lin_hbm, boxes_hbm, scale_hbm,
             lab_out, box_out, sc_out,
             rm_v, t_v, scale_v, cand_v, cand8_v, rows_a, rows_b, cv_v, ci_v,
             sv_v, si_v, lab_v, sc_v, qg_v, bl_v, bx_v, bxo_v, sem_a, sem_b,
             *, nb, q, c, gp):
    wid = lax.axis_index("s") * 2 + lax.axis_index("c")
    g_per_b = q * c // OCT
    n_r8 = CAP_OCT * 8
    n_ch = n_r8 // 128

    @pl.when(wid < nb)
    def _():
        b = wid
        pltpu.sync_copy(octmax_hbm.at[b], rm_v)
        pltpu.sync_copy(thresh_hbm.at[b], t_v)
        pltpu.sync_copy(scale_hbm.at[b], scale_v)
        t_vec = t_v[...]
        iota = _iota16()
        oct0 = b * g_per_b
        row0 = b * q

        for i in range(CAP_OCT // LANES):
            cand_v[pl.ds(i * LANES, LANES)] = _bcast(oct0)
        neg_inf = jnp.full((LANES,), -jnp.inf, jnp.float32)
        for i in range(CAP_C // LANES):
            cv_v[pl.ds(i * LANES, LANES)] = neg_inf
            ci_v[pl.ds(i * LANES, LANES)] = _bcast(0)

        def coct(i, off):
            v = rm_v[pl.ds(i * LANES, LANES)]
            m = v >= t_vec
            mi = m.astype(jnp.int32)
            pos = plsc.cumsum(mi) - mi
            dst = jnp.minimum(off + pos, CAP_OCT - 1)
            plsc.store_scatter(cand_v, [dst], oct0 + i * LANES + iota, mask=m)
            return off + plsc.all_reduce_population_count(m)[0]

        n_oct = lax.fori_loop(0, gp // LANES, coct, jnp.int32(0))

        def expand(t, _):
            j = t * LANES + iota
            o = plsc.load_gather(cand_v, [j >> 3])
            r = (o - oct0) * 8 + row0 + (j & 7)
            pad_r = row0 + (j & 8191)
            cand8_v[pl.ds(t * LANES, LANES)] = jnp.where(j < n_oct * 8, r, pad_r)
            return 0

        lax.fori_loop(0, n_r8 // LANES, expand, 0)

        bufs = (rows_a, rows_b)
        sems = (sem_a, sem_b)

        def fire(ch):
            return pltpu.async_copy(
                lin_hbm.at[cand8_v.at[pl.ds(ch * 128, 128)]],
                bufs[ch % 2], sems[ch % 2])

        cps = {0: fire(0)}
        off2 = jnp.int32(0)
        for ch in range(n_ch):
            if ch + 1 < n_ch:
                cps[ch + 1] = fire(ch + 1)
            cps[ch].wait()
            buf = bufs[ch % 2]

            def ext(j, o2, _ch=ch, _buf=buf):
                r = plsc.load_gather(cand8_v, [_bcast(_ch * 128 + j)])
                base = (r - row0) * c
                vs, ms, cnts, poss = [], [], [], []
                for k in range(c // LANES):
                    v = plsc.load_gather(_buf, [_bcast(j), k * LANES + iota])
                    m = v >= t_vec
                    mi = m.astype(jnp.int32)
                    vs.append(v)
                    ms.append(m)
                    cnts.append(plsc.all_reduce_population_count(m)[0])
                    poss.append(plsc.cumsum(mi) - mi)
                off = o2
                for k in range(c // LANES):
                    dst = jnp.minimum(off + poss[k], CAP_C - 1)
                    plsc.store_scatter(cv_v, [dst], vs[k], mask=ms[k])
                    plsc.store_scatter(ci_v, [dst], base + k * LANES + iota, mask=ms[k])
                    off = off + cnts[k]
                return off

            nj = jnp.clip(n_oct * 8 - ch * 128, 0, 128)
            off2 = lax.fori_loop(0, nj, ext, off2)
        n_c = off2

        n_cv = (n_c + LANES - 1) // LANES

        def rank_iv(iv, _):
            sl = iv * LANES + iota
            vi = plsc.load_gather(cv_v, [sl])
            xi = plsc.load_gather(ci_v, [sl])

            def inner(jv, acc):
                jsl = jv * LANES + iota
                vj = plsc.load_gather(cv_v, [jsl])
                xj = plsc.load_gather(ci_v, [jsl])
                dn = lax.GatherDimensionNumbers(
                    offset_dims=(), collapsed_slice_dims=(0,), start_index_map=(0,))
                for l in range(LANES):
                    lane = _bcast(l)[:, None]
                    vjb = lax.gather(vj, lane, dn, (1,),
                                     mode=lax.GatherScatterMode.PROMISE_IN_BOUNDS)
                    xjb = lax.gather(xj, lane, dn, (1,),
                                     mode=lax.GatherScatterMode.PROMISE_IN_BOUNDS)
                    beat = (vjb > vi) | ((vjb == vi) & (xjb < xi))
                    acc = acc + beat.astype(jnp.int32)
                return acc

            rank = lax.fori_loop(0, n_cv, inner, jnp.zeros((LANES,), jnp.int32))
            m = (rank < OUT_PAD) & (sl < n_c)
            plsc.store_scatter(sv_v, [rank], vi, mask=m)
            plsc.store_scatter(si_v, [rank], xi, mask=m)
            return 0

        lax.fori_loop(0, n_cv, rank_iv, 0)

        for s in range(OUT_PAD // LANES):
            sl = pl.ds(s * LANES, LANES)
            v = sv_v[sl]
            idx = si_v[sl]
            lab_v[sl] = idx - (idx // c) * c
            sc_v[sl] = 1.0 / (1.0 + jnp.exp(-v))
            qq = jnp.clip(idx // c, 0, q - 1)
            bflat = (row0 + qq) * 4
            qg_v[sl] = bflat >> 7
            bl_v[sl] = bflat & 127
        for s in range(OUT_PAD // LANES, 384 // LANES):
            qg_v[pl.ds(s * LANES, LANES)] = _bcast((row0 * 4) >> 7)

        copies = []
        for ch in range(384 // 128):
            copies.append(pltpu.async_copy(
                boxes_hbm.at[qg_v.at[pl.ds(ch * 128, 128)]],
                bx_v.at[pl.ds(ch * 128, 128)], sem_a))
        for cp in copies:
            cp.wait()

        box_of_lane = iota >> 2
        par = iota & 1
        sign = jnp.where((iota & 3) < 2, jnp.float32(-0.5), jnp.float32(0.5))
        sc_scale = scale_v[...]
        for g in range(OUT_PAD // 4):
            slot = 4 * g + box_of_lane
            blane = plsc.load_gather(bl_v, [slot])
            ctr = plsc.load_gather(bx_v, [slot, blane + par])
            ext2 = plsc.load_gather(bx_v, [slot, blane + 2 + par])
            bxo_v[pl.ds(g * LANES, LANES)] = (ctr + sign * ext2) * sc_scale

        pltpu.sync_copy(lab_v, lab_out.at[b])
        pltpu.sync_copy(sc_v, sc_out.at[b])
        pltpu.sync_copy(bxo_v, box_out.at[b])


def kernel(pred_logits, pred_boxes, orig_target_sizes):
    nb, q, c = pred_logits.shape
    qb = 2000
    nj = q // qb
    g_per_b = q * c // OCT
    gp = 2560

    octmax8, lin8 = pl.pallas_call(
        _sum_body,
        grid=(nb, nj),
        in_specs=[pl.BlockSpec((1, qb, c), lambda b, j: (b, j, 0))],
        out_specs=[
            pl.BlockSpec((1, 1, 1, qb // 8), lambda b, j: (b, j, 0, 0)),
            pl.BlockSpec((1, 1, qb, 128), lambda b, j: (b, j, 0, 0)),
        ],
        out_shape=[
            jax.ShapeDtypeStruct((nb, nj, 1, qb // 8), jnp.float32),
            jax.ShapeDtypeStruct((nb, nj, qb, 128), jnp.float32),
        ],
    )(pred_logits)

    thresh = pl.pallas_call(
        functools.partial(_thresh_body, nb=nb),
        out_shape=jax.ShapeDtypeStruct((nb, LANES), jnp.float32),
    )(octmax8)

    octmax = jnp.pad(octmax8.reshape(nb, g_per_b), ((0, 0), (0, gp - g_per_b)),
                     constant_values=-jnp.inf)
    lin = lin8.reshape(nb * q, 128)
    boxes128 = pred_boxes.reshape(nb * q * 4 // 128, 128)
    scale16 = jnp.tile(orig_target_sizes.astype(jnp.float32), (1, 8))

    mesh = plsc.VectorSubcoreMesh(core_axis_name="c", subcore_axis_name="s")
    sc = functools.partial(
        pl.kernel,
        out_type=[
            jax.ShapeDtypeStruct((nb, OUT_PAD), jnp.int32),
            jax.ShapeDtypeStruct((nb, OUT_PAD * 4), jnp.float32),
            jax.ShapeDtypeStruct((nb, OUT_PAD), jnp.float32),
        ],
        mesh=mesh,
        compiler_params=pltpu.CompilerParams(needs_layout_passes=False, use_tc_tiling_on_sc=True),
        scratch_types=[
            pltpu.VMEM((gp,), jnp.float32),
            pltpu.VMEM((LANES,), jnp.float32),
            pltpu.VMEM((LANES,), jnp.float32),
            pltpu.VMEM((CAP_OCT,), jnp.int32),
            pltpu.VMEM((CAP_OCT * 8,), jnp.int32),
            pltpu.VMEM((128, 128), jnp.float32),
            pltpu.VMEM((128, 128), jnp.float32),
            pltpu.VMEM((CAP_C,), jnp.float32),
            pltpu.VMEM((CAP_C,), jnp.int32),
            pltpu.VMEM((OUT_PAD,), jnp.float32),
            pltpu.VMEM((OUT_PAD,), jnp.int32),
            pltpu.VMEM((OUT_PAD,), jnp.int32),
            pltpu.VMEM((OUT_PAD,), jnp.float32),
            pltpu.VMEM((384,), jnp.int32),
            pltpu.VMEM((OUT_PAD,), jnp.int32),
            pltpu.VMEM((384, 128), jnp.float32),
            pltpu.VMEM((OUT_PAD * 4,), jnp.float32),
            pltpu.SemaphoreType.DMA,
            pltpu.SemaphoreType.DMA,
        ],
    )(functools.partial(_sc_body, nb=nb, q=q, c=c, gp=gp))

    labels_p, boxes_p, scores_p = sc(octmax, thresh, lin, boxes128, scale16)
    return (labels_p[:, :NUM_TOP],
            boxes_p.reshape(nb, OUT_PAD, 4)[:, :NUM_TOP],
            scores_p[:, :NUM_TOP])


# speedup vs baseline: 1.3668x; 1.0919x over previous
import functools

import jax
import jax.numpy as jnp
from jax import lax
from jax.experimental import pallas as pl
from jax.experimental.pallas import tpu as pltpu
from jax.experimental.pallas import tpu_sc as plsc

NUM_TOP = 300
OUT_PAD = 304
OCT = 640
CAP_OCT = 384
CAP_C = 512
LANES = 16

_I32_FLIP = 0x7FFFFFFF


def _sum_body(x_ref, om_ref, lin_ref):
    x = x_ref[0]
    qb, c = x.shape
    om_ref[0, 0, 0, :] = jnp.max(x.reshape(qb // 8, 8, c), axis=(1, 2))
    pad = jnp.full((qb, 128 - c), -jnp.inf, jnp.float32)
    lin_ref[0, 0] = jnp.concatenate([x, pad], axis=-1)


def _thresh_body(rm_ref, t_ref, *, nb):
    rm = rm_ref[...]
    u = lax.bitcast_convert_type(rm, jnp.int32)
    key = jnp.where(u >= 0, u, u ^ jnp.int32(_I32_FLIP))

    def it(_, lohi):
        lo, hi = lohi
        fl = (lo >> 1) + (hi >> 1) + (lo & hi & 1)
        mid = fl + ((lo ^ hi) & 1)
        cnt = jnp.sum((key >= mid).astype(jnp.int32), axis=(1, 2, 3), keepdims=True)
        ok = cnt >= NUM_TOP
        return jnp.where(ok, mid, lo), jnp.where(ok, hi, mid - 1)

    lo0 = jnp.full((nb, 1, 1, 1), jnp.iinfo(jnp.int32).min, jnp.int32)
    hi0 = jnp.full((nb, 1, 1, 1), jnp.iinfo(jnp.int32).max, jnp.int32)
    k_fin, _ = lax.fori_loop(0, 32, it, (lo0, hi0))
    ub = jnp.where(k_fin >= 0, k_fin, k_fin ^ jnp.int32(_I32_FLIP))
    t = lax.bitcast_convert_type(ub, jnp.float32)
    t_ref[...] = jnp.broadcast_to(t.reshape(nb, 1), (nb, LANES))


def _iota16():
    return lax.iota(jnp.int32, LANES)


def _bcast(x):
    return jnp.full((LANES,), x, jnp.int32)


def _sc_body(octmax_hbm, thresh_hbm, lin_hbm, boxes_hbm, scale_hbm,
             lab_out, box_out, sc_out,
             rm_v, t_v, scale_v, cand_v, cand8_v, rows_a, rows_b, cv_v, ci_v,
             sv_v, si_v, lab_v, sc_v, qg_v, bl_v, bx_v, bxo_v, sem_a, sem_b,
             *, nb, q, c, gp):
    wid = lax.axis_index("s") * 2 + lax.axis_index("c")
    g_per_b = q * c // OCT
    n_r8 = CAP_OCT * 8
    n_ch = n_r8 // 128

    @pl.when(wid < nb)
    def _():
        b = wid
        pltpu.sync_copy(octmax_hbm.at[b], rm_v)
        pltpu.sync_copy(thresh_hbm.at[b], t_v)
        pltpu.sync_copy(scale_hbm.at[b], scale_v)
        t_vec = t_v[...]
        iota = _iota16()
        oct0 = b * g_per_b
        row0 = b * q

        for i in range(CAP_OCT // LANES):
            cand_v[pl.ds(i * LANES, LANES)] = _bcast(oct0)
        neg_inf = jnp.full((LANES,), -jnp.inf, jnp.float32)
        for i in range(CAP_C // LANES):
            cv_v[pl.ds(i * LANES, LANES)] = neg_inf
            ci_v[pl.ds(i * LANES, LANES)] = _bcast(0)

        def coct(i, off):
            v = rm_v[pl.ds(i * LANES, LANES)]
            m = v >= t_vec
            mi = m.astype(jnp.int32)
            pos = plsc.cumsum(mi) - mi
            dst = jnp.minimum(off + pos, CAP_OCT - 1)
            plsc.store_scatter(cand_v, [dst], oct0 + i * LANES + iota, mask=m)
            return off + plsc.all_reduce_population_count(m)[0]

        n_oct = lax.fori_loop(0, gp // LANES, coct, jnp.int32(0))

        def expand(t, _):
            j = t * LANES + iota
            o = plsc.load_gather(cand_v, [j >> 3])
            r = (o - oct0) * 8 + row0 + (j & 7)
            pad_r = row0 + (j & 8191)
            cand8_v[pl.ds(t * LANES, LANES)] = jnp.where(j < n_oct * 8, r, pad_r)
            return 0

        lax.fori_loop(0, n_r8 // LANES, expand, 0)

        bufs = (rows_a, rows_b)
        sems = (sem_a, sem_b)

        def fire(ch):
            return pltpu.async_copy(
                lin_hbm.at[cand8_v.at[pl.ds(ch * 128, 128)]],
                bufs[ch % 2], sems[ch % 2])

        cps = {0: fire(0)}
        off2 = jnp.int32(0)
        for ch in range(n_ch):
            if ch + 1 < n_ch:
                cps[ch + 1] = fire(ch + 1)
            cps[ch].wait()
            buf = bufs[ch % 2]

            def ext(j, o2, _ch=ch, _buf=buf):
                r = plsc.load_gather(cand8_v, [_bcast(_ch * 128 + j)])
                base = (r - row0) * c
                vs, ms, cnts, poss = [], [], [], []
                for k in range(c // LANES):
                    v = plsc.load_gather(_buf, [_bcast(j), k * LANES + iota])
                    m = v >= t_vec
                    mi = m.astype(jnp.int32)
                    vs.append(v)
                    ms.append(m)
                    cnts.append(plsc.all_reduce_population_count(m)[0])
                    poss.append(plsc.cumsum(mi) - mi)
                off = o2
                for k in range(c // LANES):
                    dst = jnp.minimum(off + poss[k], CAP_C - 1)
                    plsc.store_scatter(cv_v, [dst], vs[k], mask=ms[k])
                    plsc.store_scatter(ci_v, [dst], base + k * LANES + iota, mask=ms[k])
                    off = off + cnts[k]
                return off

            nj = jnp.clip(n_oct * 8 - ch * 128, 0, 128)
            off2 = lax.fori_loop(0, nj, ext, off2)
        n_c = off2

        n_cv = (n_c + LANES - 1) // LANES

        def rank_iv(iv, _):
            sl = iv * LANES + iota
            vi = plsc.load_gather(cv_v, [sl])
            xi = plsc.load_gather(ci_v, [sl])

            def inner(jv, acc):
                jsl = jv * LANES + iota
                vj = plsc.load_gather(cv_v, [jsl])
                xj = plsc.load_gather(ci_v, [jsl])
                dn = lax.GatherDimensionNumbers(
                    offset_dims=(), collapsed_slice_dims=(0,), start_index_map=(0,))
                for l in range(LANES):
                    lane = _bcast(l)[:, None]
                    vjb = lax.gather(vj, lane, dn, (1,),
                                     mode=lax.GatherScatterMode.PROMISE_IN_BOUNDS)
                    xjb = lax.gather(xj, lane, dn, (1,),
                                     mode=lax.GatherScatterMode.PROMISE_IN_BOUNDS)
                    beat = (vjb > vi) | ((vjb == vi) & (xjb < xi))
                    acc = acc + beat.astype(jnp.int32)
                return acc

            rank = lax.fori_loop(0, n_cv, inner, jnp.zeros((LANES,), jnp.int32))
            m = (rank < OUT_PAD) & (sl < n_c)
            plsc.store_scatter(sv_v, [rank], vi, mask=m)
            plsc.store_scatter(si_v, [rank], xi, mask=m)
            return 0

        lax.fori_loop(0, n_cv, rank_iv, 0)

        for s in range(OUT_PAD // LANES):
            sl = pl.ds(s * LANES, LANES)
            v = sv_v[sl]
            idx = si_v[sl]
            lab_v[sl] = idx - (idx // c) * c
            sc_v[sl] = 1.0 / (1.0 + jnp.exp(-v))
            qq = jnp.clip(idx // c, 0, q - 1)
            bflat = (row0 + qq) * 4
            qg_v[sl] = bflat >> 7
            bl_v[sl] = bflat & 127
        for s in range(OUT_PAD // LANES, 384 // LANES):
            qg_v[pl.ds(s * LANES, LANES)] = _bcast((row0 * 4) >> 7)

        copies = []
        for ch in range(384 // 128):
            copies.append(pltpu.async_copy(
                boxes_hbm.at[qg_v.at[pl.ds(ch * 128, 128)]],
                bx_v.at[pl.ds(ch * 128, 128)], sem_a))
        for cp in copies:
            cp.wait()

        box_of_lane = iota >> 2
        par = iota & 1
        sign = jnp.where((iota & 3) < 2, jnp.float32(-0.5), jnp.float32(0.5))
        sc_scale = scale_v[...]
        for g in range(OUT_PAD // 4):
            slot = 4 * g + box_of_lane
            blane = plsc.load_gather(bl_v, [slot])
            ctr = plsc.load_gather(bx_v, [slot, blane + par])
            ext2 = plsc.load_gather(bx_v, [slot, blane + 2 + par])
            bxo_v[pl.ds(g * LANES, LANES)] = (ctr + sign * ext2) * sc_scale

        pltpu.sync_copy(lab_v, lab_out.at[b])
        pltpu.sync_copy(sc_v, sc_out.at[b])
        pltpu.sync_copy(bxo_v, box_out.at[b])


def kernel(pred_logits, pred_boxes, orig_target_sizes):
    nb, q, c = pred_logits.shape
    qb = 4000
    nj = q // qb
    g_per_b = q * c // OCT
    gp = 2560

    octmax8, lin8 = pl.pallas_call(
        _sum_body,
        grid=(nb, nj),
        in_specs=[pl.BlockSpec((1, qb, c), lambda b, j: (b, j, 0))],
        out_specs=[
            pl.BlockSpec((1, 1, 1, qb // 8), lambda b, j: (b, j, 0, 0)),
            pl.BlockSpec((1, 1, qb, 128), lambda b, j: (b, j, 0, 0)),
        ],
        out_shape=[
            jax.ShapeDtypeStruct((nb, nj, 1, qb // 8), jnp.float32),
            jax.ShapeDtypeStruct((nb, nj, qb, 128), jnp.float32),
        ],
    )(pred_logits)

    thresh = pl.pallas_call(
        functools.partial(_thresh_body, nb=nb),
        out_shape=jax.ShapeDtypeStruct((nb, LANES), jnp.float32),
    )(octmax8)

    octmax = jnp.pad(octmax8.reshape(nb, g_per_b), ((0, 0), (0, gp - g_per_b)),
                     constant_values=-jnp.inf)
    lin = lin8.reshape(nb * q, 128)
    boxes128 = pred_boxes.reshape(nb * q * 4 // 128, 128)
    scale16 = jnp.tile(orig_target_sizes.astype(jnp.float32), (1, 8))

    mesh = plsc.VectorSubcoreMesh(core_axis_name="c", subcore_axis_name="s")
    sc = functools.partial(
        pl.kernel,
        out_type=[
            jax.ShapeDtypeStruct((nb, OUT_PAD), jnp.int32),
            jax.ShapeDtypeStruct((nb, OUT_PAD * 4), jnp.float32),
            jax.ShapeDtypeStruct((nb, OUT_PAD), jnp.float32),
        ],
        mesh=mesh,
        compiler_params=pltpu.CompilerParams(needs_layout_passes=False, use_tc_tiling_on_sc=True),
        scratch_types=[
            pltpu.VMEM((gp,), jnp.float32),
            pltpu.VMEM((LANES,), jnp.float32),
            pltpu.VMEM((LANES,), jnp.float32),
            pltpu.VMEM((CAP_OCT,), jnp.int32),
            pltpu.VMEM((CAP_OCT * 8,), jnp.int32),
            pltpu.VMEM((128, 128), jnp.float32),
            pltpu.VMEM((128, 128), jnp.float32),
            pltpu.VMEM((CAP_C,), jnp.float32),
            pltpu.VMEM((CAP_C,), jnp.int32),
            pltpu.VMEM((OUT_PAD,), jnp.float32),
            pltpu.VMEM((OUT_PAD,), jnp.int32),
            pltpu.VMEM((OUT_PAD,), jnp.int32),
            pltpu.VMEM((OUT_PAD,), jnp.float32),
            pltpu.VMEM((384,), jnp.int32),
            pltpu.VMEM((OUT_PAD,), jnp.int32),
            pltpu.VMEM((384, 128), jnp.float32),
            pltpu.VMEM((OUT_PAD * 4,), jnp.float32),
            pltpu.SemaphoreType.DMA,
            pltpu.SemaphoreType.DMA,
        ],
    )(functools.partial(_sc_body, nb=nb, q=q, c=c, gp=gp))

    labels_p, boxes_p, scores_p = sc(octmax, thresh, lin, boxes128, scale16)
    return (labels_p[:, :NUM_TOP],
            boxes_p.reshape(nb, OUT_PAD, 4)[:, :NUM_TOP],
            scores_p[:, :NUM_TOP])


# speedup vs baseline: 1.4212x; 1.0398x over previous
import functools

import jax
import jax.numpy as jnp
from jax import lax
from jax.experimental import pallas as pl
from jax.experimental.pallas import tpu as pltpu
from jax.experimental.pallas import tpu_sc as plsc

NUM_TOP = 300
OUT_PAD = 304
OCT = 640
CAP_OCT = 384
CAP_C = 512
LANES = 16

_I32_FLIP = 0x7FFFFFFF


def _sum_body(x_ref, om_ref, lin_ref):
    x = x_ref[0]
    qb, c = x.shape
    om_ref[0, 0, 0, :] = jnp.max(x.reshape(qb // 8, 8, c), axis=(1, 2))
    pad = jnp.full((qb, 128 - c), -jnp.inf, jnp.float32)
    lin_ref[0, 0] = jnp.concatenate([x, pad], axis=-1)


def _thresh_body(rm_ref, t_ref, *, nb):
    rm = rm_ref[...]
    u = lax.bitcast_convert_type(rm, jnp.int32)
    key = jnp.where(u >= 0, u, u ^ jnp.int32(_I32_FLIP))

    def it(_, lohi):
        lo, hi = lohi
        fl = (lo >> 1) + (hi >> 1) + (lo & hi & 1)
        mid = fl + ((lo ^ hi) & 1)
        cnt = jnp.sum((key >= mid).astype(jnp.int32), axis=(1, 2, 3), keepdims=True)
        ok = cnt >= NUM_TOP
        return jnp.where(ok, mid, lo), jnp.where(ok, hi, mid - 1)

    lo0 = jnp.full((nb, 1, 1, 1), jnp.iinfo(jnp.int32).min, jnp.int32)
    hi0 = jnp.full((nb, 1, 1, 1), jnp.iinfo(jnp.int32).max, jnp.int32)
    k_fin, _ = lax.fori_loop(0, 32, it, (lo0, hi0))
    ub = jnp.where(k_fin >= 0, k_fin, k_fin ^ jnp.int32(_I32_FLIP))
    t = lax.bitcast_convert_type(ub, jnp.float32)
    t_ref[...] = jnp.broadcast_to(t.reshape(nb, 1), (nb, LANES))


def _iota16():
    return lax.iota(jnp.int32, LANES)


def _bcast(x):
    return jnp.full((LANES,), x, jnp.int32)


def _sc_body(octmax_hbm, thresh_hbm, lin_hbm, boxes_hbm, scale_hbm,
             lab_out, box_out, sc_out,
             rm_v, t_v, scale_v, cand_v, cand8_v, rows_a, rows_b, cv_v, ci_v,
             sv_v, si_v, lab_v, sc_v, qg_v, bl_v, bx_v, bxo_v, sem_a, sem_b,
             *, nb, q, c, gp):
    wid = lax.axis_index("s") * 2 + lax.axis_index("c")
    g_per_b = q * c // OCT
    n_r8 = CAP_OCT * 8
    n_ch = n_r8 // 128

    @pl.when(wid < nb)
    def _():
        b = wid
        pltpu.sync_copy(octmax_hbm.at[b], rm_v)
        pltpu.sync_copy(thresh_hbm.at[b], t_v)
        pltpu.sync_copy(scale_hbm.at[b], scale_v)
        t_vec = t_v[...]
        iota = _iota16()
        oct0 = b * g_per_b
        row0 = b * q

        for i in range(CAP_OCT // LANES):
            cand_v[pl.ds(i * LANES, LANES)] = _bcast(oct0)
        neg_inf = jnp.full((LANES,), -jnp.inf, jnp.float32)
        for i in range(CAP_C // LANES):
            cv_v[pl.ds(i * LANES, LANES)] = neg_inf
            ci_v[pl.ds(i * LANES, LANES)] = _bcast(0)

        def coct(i, off):
            v = rm_v[pl.ds(i * LANES, LANES)]
            m = v >= t_vec
            mi = m.astype(jnp.int32)
            pos = plsc.cumsum(mi) - mi
            dst = jnp.minimum(off + pos, CAP_OCT - 1)
            plsc.store_scatter(cand_v, [dst], oct0 + i * LANES + iota, mask=m)
            return off + plsc.all_reduce_population_count(m)[0]

        n_oct = lax.fori_loop(0, gp // LANES, coct, jnp.int32(0))

        def expand(t, _):
            j = t * LANES + iota
            o = plsc.load_gather(cand_v, [j >> 3])
            r = (o - oct0) * 8 + row0 + (j & 7)
            pad_r = row0 + (j & 8191)
            cand8_v[pl.ds(t * LANES, LANES)] = jnp.where(j < n_oct * 8, r, pad_r)
            return 0

        lax.fori_loop(0, n_r8 // LANES, expand, 0)

        bufs = (rows_a, rows_b)
        sems = (sem_a, sem_b)

        def fire(ch):
            return pltpu.async_copy(
                lin_hbm.at[cand8_v.at[pl.ds(ch * 128, 128)]],
                bufs[ch % 2], sems[ch % 2])

        cps = {0: fire(0)}
        off2 = jnp.int32(0)
        for ch in range(n_ch):
            if ch + 1 < n_ch:
                cps[ch + 1] = fire(ch + 1)
            cps[ch].wait()
            buf = bufs[ch % 2]

            def ext(j, o2, _ch=ch, _buf=buf):
                r = plsc.load_gather(cand8_v, [_bcast(_ch * 128 + j)])
                base = (r - row0) * c
                vs, ms, cnts, poss = [], [], [], []
                for k in range(c // LANES):
                    v = plsc.load_gather(_buf, [_bcast(j), k * LANES + iota])
                    m = v >= t_vec
                    mi = m.astype(jnp.int32)
                    vs.append(v)
                    ms.append(m)
                    cnts.append(plsc.all_reduce_population_count(m)[0])
                    poss.append(plsc.cumsum(mi) - mi)
                off = o2
                for k in range(c // LANES):
                    dst = jnp.minimum(off + poss[k], CAP_C - 1)
                    plsc.store_scatter(cv_v, [dst], vs[k], mask=ms[k])
                    plsc.store_scatter(ci_v, [dst], base + k * LANES + iota, mask=ms[k])
                    off = off + cnts[k]
                return off

            nj = jnp.clip(n_oct * 8 - ch * 128, 0, 128)
            off2 = lax.fori_loop(0, nj, ext, off2)
        n_c = off2

        n_cv = (n_c + LANES - 1) // LANES

        def rank_iv(iv, _):
            sl = iv * LANES + iota
            vi = plsc.load_gather(cv_v, [sl])
            xi = plsc.load_gather(ci_v, [sl])

            def inner(jv, acc):
                jsl = jv * LANES + iota
                vj = plsc.load_gather(cv_v, [jsl])
                xj = plsc.load_gather(ci_v, [jsl])
                dn = lax.GatherDimensionNumbers(
                    offset_dims=(), collapsed_slice_dims=(0,), start_index_map=(0,))
                for l in range(LANES):
                    lane = _bcast(l)[:, None]
                    vjb = lax.gather(vj, lane, dn, (1,),
                                     mode=lax.GatherScatterMode.PROMISE_IN_BOUNDS)
                    xjb = lax.gather(xj, lane, dn, (1,),
                                     mode=lax.GatherScatterMode.PROMISE_IN_BOUNDS)
                    beat = (vjb > vi) | ((vjb == vi) & (xjb < xi))
                    acc = acc + beat.astype(jnp.int32)
                return acc

            rank = lax.fori_loop(0, n_cv, inner, jnp.zeros((LANES,), jnp.int32))
            m = (rank < OUT_PAD) & (sl < n_c)
            plsc.store_scatter(sv_v, [rank], vi, mask=m)
            plsc.store_scatter(si_v, [rank], xi, mask=m)
            return 0

        lax.fori_loop(0, n_cv, rank_iv, 0)

        for s in range(OUT_PAD // LANES):
            sl = pl.ds(s * LANES, LANES)
            v = sv_v[sl]
            idx = si_v[sl]
            lab_v[sl] = idx - (idx // c) * c
            sc_v[sl] = 1.0 / (1.0 + jnp.exp(-v))
            qq = jnp.clip(idx // c, 0, q - 1)
            bflat = (row0 + qq) * 4
            qg_v[sl] = bflat >> 7
            bl_v[sl] = bflat & 127
        for s in range(OUT_PAD // LANES, 384 // LANES):
            qg_v[pl.ds(s * LANES, LANES)] = _bcast((row0 * 4) >> 7)

        copies = []
        for ch in range(384 // 128):
            copies.append(pltpu.async_copy(
                boxes_hbm.at[qg_v.at[pl.ds(ch * 128, 128)]],
                bx_v.at[pl.ds(ch * 128, 128)], sem_a))
        for cp in copies:
            cp.wait()

        box_of_lane = iota >> 2
        par = iota & 1
        sign = jnp.where((iota & 3) < 2, jnp.float32(-0.5), jnp.float32(0.5))
        sc_scale = scale_v[...]
        for g in range(OUT_PAD // 4):
            slot = 4 * g + box_of_lane
            blane = plsc.load_gather(bl_v, [slot])
            ctr = plsc.load_gather(bx_v, [slot, blane + par])
            ext2 = plsc.load_gather(bx_v, [slot, blane + 2 + par])
            bxo_v[pl.ds(g * LANES, LANES)] = (ctr + sign * ext2) * sc_scale

        pltpu.sync_copy(lab_v, lab_out.at[b])
        pltpu.sync_copy(sc_v, sc_out.at[b])
        pltpu.sync_copy(bxo_v, box_out.at[b])


def kernel(pred_logits, pred_boxes, orig_target_sizes):
    nb, q, c = pred_logits.shape
    qb = 10000
    nj = q // qb
    g_per_b = q * c // OCT
    gp = 2560

    octmax8, lin8 = pl.pallas_call(
        _sum_body,
        grid=(nb, nj),
        in_specs=[pl.BlockSpec((1, qb, c), lambda b, j: (b, j, 0))],
        out_specs=[
            pl.BlockSpec((1, 1, 1, qb // 8), lambda b, j: (b, j, 0, 0)),
            pl.BlockSpec((1, 1, qb, 128), lambda b, j: (b, j, 0, 0)),
        ],
        out_shape=[
            jax.ShapeDtypeStruct((nb, nj, 1, qb // 8), jnp.float32),
            jax.ShapeDtypeStruct((nb, nj, qb, 128), jnp.float32),
        ],
    )(pred_logits)

    thresh = pl.pallas_call(
        functools.partial(_thresh_body, nb=nb),
        out_shape=jax.ShapeDtypeStruct((nb, LANES), jnp.float32),
    )(octmax8)

    octmax = jnp.pad(octmax8.reshape(nb, g_per_b), ((0, 0), (0, gp - g_per_b)),
                     constant_values=-jnp.inf)
    lin = lin8.reshape(nb * q, 128)
    boxes128 = pred_boxes.reshape(nb * q * 4 // 128, 128)
    scale16 = jnp.tile(orig_target_sizes.astype(jnp.float32), (1, 8))

    mesh = plsc.VectorSubcoreMesh(core_axis_name="c", subcore_axis_name="s")
    sc = functools.partial(
        pl.kernel,
        out_type=[
            jax.ShapeDtypeStruct((nb, OUT_PAD), jnp.int32),
            jax.ShapeDtypeStruct((nb, OUT_PAD * 4), jnp.float32),
            jax.ShapeDtypeStruct((nb, OUT_PAD), jnp.float32),
        ],
        mesh=mesh,
        compiler_params=pltpu.CompilerParams(needs_layout_passes=False, use_tc_tiling_on_sc=True),
        scratch_types=[
            pltpu.VMEM((gp,), jnp.float32),
            pltpu.VMEM((LANES,), jnp.float32),
            pltpu.VMEM((LANES,), jnp.float32),
            pltpu.VMEM((CAP_OCT,), jnp.int32),
            pltpu.VMEM((CAP_OCT * 8,), jnp.int32),
            pltpu.VMEM((128, 128), jnp.float32),
            pltpu.VMEM((128, 128), jnp.float32),
            pltpu.VMEM((CAP_C,), jnp.float32),
            pltpu.VMEM((CAP_C,), jnp.int32),
            pltpu.VMEM((OUT_PAD,), jnp.float32),
            pltpu.VMEM((OUT_PAD,), jnp.int32),
            pltpu.VMEM((OUT_PAD,), jnp.int32),
            pltpu.VMEM((OUT_PAD,), jnp.float32),
            pltpu.VMEM((384,), jnp.int32),
            pltpu.VMEM((OUT_PAD,), jnp.int32),
            pltpu.VMEM((384, 128), jnp.float32),
            pltpu.VMEM((OUT_PAD * 4,), jnp.float32),
            pltpu.SemaphoreType.DMA,
            pltpu.SemaphoreType.DMA,
        ],
    )(functools.partial(_sc_body, nb=nb, q=q, c=c, gp=gp))

    labels_p, boxes_p, scores_p = sc(octmax, thresh, lin, boxes128, scale16)
    return (labels_p[:, :NUM_TOP],
            boxes_p.reshape(nb, OUT_PAD, 4)[:, :NUM_TOP],
            scores_p[:, :NUM_TOP])


# speedup vs baseline: 1.4234x; 1.0016x over previous
import functools

import jax
import jax.numpy as jnp
from jax import lax
from jax.experimental import pallas as pl
from jax.experimental.pallas import tpu as pltpu
from jax.experimental.pallas import tpu_sc as plsc

NUM_TOP = 300
OUT_PAD = 304
OCT = 640
CAP_OCT = 384
CAP_C = 512
LANES = 16

_I32_FLIP = 0x7FFFFFFF


def _sum_body(x_ref, om_ref, lin_ref):
    x = x_ref[0]
    qb, c = x.shape
    om_ref[0, 0, 0, :] = jnp.max(x.reshape(qb // 8, 8, c), axis=(1, 2))
    pad = jnp.full((qb, 128 - c), -jnp.inf, jnp.float32)
    lin_ref[0, 0] = jnp.concatenate([x, pad], axis=-1)


def _thresh_body(rm_ref, t_ref, *, nb):
    rm = rm_ref[...]
    u = lax.bitcast_convert_type(rm, jnp.int32)
    key = jnp.where(u >= 0, u, u ^ jnp.int32(_I32_FLIP))

    def it(_, lohi):
        lo, hi = lohi
        fl = (lo >> 1) + (hi >> 1) + (lo & hi & 1)
        mid = fl + ((lo ^ hi) & 1)
        cnt = jnp.sum((key >= mid).astype(jnp.int32), axis=(1, 2, 3), keepdims=True)
        ok = cnt >= NUM_TOP
        return jnp.where(ok, mid, lo), jnp.where(ok, hi, mid - 1)

    lo0 = jnp.full((nb, 1, 1, 1), jnp.iinfo(jnp.int32).min, jnp.int32)
    hi0 = jnp.full((nb, 1, 1, 1), jnp.iinfo(jnp.int32).max, jnp.int32)
    k_fin, _ = lax.fori_loop(0, 32, it, (lo0, hi0))
    ub = jnp.where(k_fin >= 0, k_fin, k_fin ^ jnp.int32(_I32_FLIP))
    t = lax.bitcast_convert_type(ub, jnp.float32)
    t_ref[...] = jnp.broadcast_to(t.reshape(nb, 1), (nb, LANES))


def _iota16():
    return lax.iota(jnp.int32, LANES)


def _bcast(x):
    return jnp.full((LANES,), x, jnp.int32)


def _sc_body(octmax_hbm, thresh_hbm, lin_hbm, boxes_hbm, scale_hbm,
             lab_out, box_out, sc_out,
             rm_v, t_v, scale_v, cand_v, cand8_v, rows_a, rows_b, cv_v, ci_v,
             sv_v, si_v, lab_v, sc_v, qg_v, bl_v, bx_v, bxo_v, sem_a, sem_b,
             *, nb, q, c, gp):
    wid = lax.axis_index("s") * 2 + lax.axis_index("c")
    g_per_b = q * c // OCT
    n_r8 = CAP_OCT * 8
    n_ch = n_r8 // 128

    @pl.when(wid < nb)
    def _():
        b = wid
        pltpu.sync_copy(octmax_hbm.at[b], rm_v)
        pltpu.sync_copy(thresh_hbm.at[b], t_v)
        pltpu.sync_copy(scale_hbm.at[b], scale_v)
        t_vec = t_v[...]
        iota = _iota16()
        oct0 = b * g_per_b
        row0 = b * q

        for i in range(CAP_OCT // LANES):
            cand_v[pl.ds(i * LANES, LANES)] = _bcast(oct0)
        neg_inf = jnp.full((LANES,), -jnp.inf, jnp.float32)
        for i in range(CAP_C // LANES):
            cv_v[pl.ds(i * LANES, LANES)] = neg_inf
            ci_v[pl.ds(i * LANES, LANES)] = _bcast(0)

        def coct(i, off):
            v = rm_v[pl.ds(i * LANES, LANES)]
            m = v >= t_vec
            mi = m.astype(jnp.int32)
            pos = plsc.cumsum(mi) - mi
            dst = jnp.minimum(off + pos, CAP_OCT - 1)
            plsc.store_scatter(cand_v, [dst], oct0 + i * LANES + iota, mask=m)
            return off + plsc.all_reduce_population_count(m)[0]

        n_oct = lax.fori_loop(0, gp // LANES, coct, jnp.int32(0))

        def expand(t, _):
            j = t * LANES + iota
            o = plsc.load_gather(cand_v, [j >> 3])
            r = (o - oct0) * 8 + row0 + (j & 7)
            pad_r = row0 + (j & 8191)
            cand8_v[pl.ds(t * LANES, LANES)] = jnp.where(j < n_oct * 8, r, pad_r)
            return 0

        lax.fori_loop(0, n_r8 // LANES, expand, 0)

        bufs = (rows_a, rows_b)
        sems = (sem_a, sem_b)

        def fire(ch):
            return pltpu.async_copy(
                lin_hbm.at[cand8_v.at[pl.ds(ch * 128, 128)]],
                bufs[ch % 2], sems[ch % 2])

        cps = {0: fire(0)}
        off2 = jnp.int32(0)
        for ch in range(n_ch):
            if ch + 1 < n_ch:
                cps[ch + 1] = fire(ch + 1)
            cps[ch].wait()
            buf = bufs[ch % 2]

            def ext(j, o2, _ch=ch, _buf=buf):
                r = plsc.load_gather(cand8_v, [_bcast(_ch * 128 + j)])
                base = (r - row0) * c
                vs, ms, cnts, poss = [], [], [], []
                for k in range(c // LANES):
                    v = plsc.load_gather(_buf, [_bcast(j), k * LANES + iota])
                    m = v >= t_vec
                    mi = m.astype(jnp.int32)
                    vs.append(v)
                    ms.append(m)
                    cnts.append(plsc.all_reduce_population_count(m)[0])
                    poss.append(plsc.cumsum(mi) - mi)
                off = o2
                for k in range(c // LANES):
                    dst = jnp.minimum(off + poss[k], CAP_C - 1)
                    plsc.store_scatter(cv_v, [dst], vs[k], mask=ms[k])
                    plsc.store_scatter(ci_v, [dst], base + k * LANES + iota, mask=ms[k])
                    off = off + cnts[k]
                return off

            nj = jnp.clip(n_oct * 8 - ch * 128, 0, 128)
            off2 = lax.fori_loop(0, nj, ext, off2)
        n_c = off2

        n_cv = (n_c + LANES - 1) // LANES

        def rank_iv(iv, _):
            sl = iv * LANES + iota
            vi = plsc.load_gather(cv_v, [sl])
            xi = plsc.load_gather(ci_v, [sl])

            def inner(jv, acc):
                jsl = jv * LANES + iota
                vj = plsc.load_gather(cv_v, [jsl])
                xj = plsc.load_gather(ci_v, [jsl])
                dn = lax.GatherDimensionNumbers(
                    offset_dims=(), collapsed_slice_dims=(0,), start_index_map=(0,))
                for l in range(LANES):
                    lane = _bcast(l)[:, None]
                    vjb = lax.gather(vj, lane, dn, (1,),
                                     mode=lax.GatherScatterMode.PROMISE_IN_BOUNDS)
                    xjb = lax.gather(xj, lane, dn, (1,),
                                     mode=lax.GatherScatterMode.PROMISE_IN_BOUNDS)
                    beat = (vjb > vi) | ((vjb == vi) & (xjb < xi))
                    acc = acc + beat.astype(jnp.int32)
                return acc

            rank = lax.fori_loop(0, n_cv, inner, jnp.zeros((LANES,), jnp.int32))
            m = (rank < OUT_PAD) & (sl < n_c)
            plsc.store_scatter(sv_v, [rank], vi, mask=m)
            plsc.store_scatter(si_v, [rank], xi, mask=m)
            return 0

        lax.fori_loop(0, n_cv, rank_iv, 0)

        for s in range(OUT_PAD // LANES):
            sl = pl.ds(s * LANES, LANES)
            v = sv_v[sl]
            idx = si_v[sl]
            lab_v[sl] = idx - (idx // c) * c
            sc_v[sl] = 1.0 / (1.0 + jnp.exp(-v))
            qq = jnp.clip(idx // c, 0, q - 1)
            bflat = (row0 + qq) * 4
            qg_v[sl] = bflat >> 7
            bl_v[sl] = bflat & 127
        for s in range(OUT_PAD // LANES, 384 // LANES):
            qg_v[pl.ds(s * LANES, LANES)] = _bcast((row0 * 4) >> 7)

        copies = []
        for ch in range(384 // 128):
            copies.append(pltpu.async_copy(
                boxes_hbm.at[qg_v.at[pl.ds(ch * 128, 128)]],
                bx_v.at[pl.ds(ch * 128, 128)], sem_a))
        for cp in copies:
            cp.wait()

        box_of_lane = iota >> 2
        par = iota & 1
        sign = jnp.where((iota & 3) < 2, jnp.float32(-0.5), jnp.float32(0.5))
        sc_scale = scale_v[...]
        for g in range(OUT_PAD // 4):
            slot = 4 * g + box_of_lane
            blane = plsc.load_gather(bl_v, [slot])
            ctr = plsc.load_gather(bx_v, [slot, blane + par])
            ext2 = plsc.load_gather(bx_v, [slot, blane + 2 + par])
            bxo_v[pl.ds(g * LANES, LANES)] = (ctr + sign * ext2) * sc_scale

        pltpu.sync_copy(lab_v, lab_out.at[b])
        pltpu.sync_copy(sc_v, sc_out.at[b])
        pltpu.sync_copy(bxo_v, box_out.at[b])


def kernel(pred_logits, pred_boxes, orig_target_sizes):
    nb, q, c = pred_logits.shape
    qb = 20000
    nj = q // qb
    g_per_b = q * c // OCT
    gp = 2560

    octmax8, lin8 = pl.pallas_call(
        _sum_body,
        grid=(nb, nj),
        in_specs=[pl.BlockSpec((1, qb, c), lambda b, j: (b, j, 0))],
        out_specs=[
            pl.BlockSpec((1, 1, 1, qb // 8), lambda b, j: (b, j, 0, 0)),
            pl.BlockSpec((1, 1, qb, 128), lambda b, j: (b, j, 0, 0)),
        ],
        out_shape=[
            jax.ShapeDtypeStruct((nb, nj, 1, qb // 8), jnp.float32),
            jax.ShapeDtypeStruct((nb, nj, qb, 128), jnp.float32),
        ],
    )(pred_logits)

    thresh = pl.pallas_call(
        functools.partial(_thresh_body, nb=nb),
        out_shape=jax.ShapeDtypeStruct((nb, LANES), jnp.float32),
    )(octmax8)

    octmax = jnp.pad(octmax8.reshape(nb, g_per_b), ((0, 0), (0, gp - g_per_b)),
                     constant_values=-jnp.inf)
    lin = lin8.reshape(nb * q, 128)
    boxes128 = pred_boxes.reshape(nb * q * 4 // 128, 128)
    scale16 = jnp.tile(orig_target_sizes.astype(jnp.float32), (1, 8))

    mesh = plsc.VectorSubcoreMesh(core_axis_name="c", subcore_axis_name="s")
    sc = functools.partial(
        pl.kernel,
        out_type=[
            jax.ShapeDtypeStruct((nb, OUT_PAD), jnp.int32),
            jax.ShapeDtypeStruct((nb, OUT_PAD * 4), jnp.float32),
            jax.ShapeDtypeStruct((nb, OUT_PAD), jnp.float32),
        ],
        mesh=mesh,
        compiler_params=pltpu.CompilerParams(needs_layout_passes=False, use_tc_tiling_on_sc=True),
        scratch_types=[
            pltpu.VMEM((gp,), jnp.float32),
            pltpu.VMEM((LANES,), jnp.float32),
            pltpu.VMEM((LANES,), jnp.float32),
            pltpu.VMEM((CAP_OCT,), jnp.int32),
            pltpu.VMEM((CAP_OCT * 8,), jnp.int32),
            pltpu.VMEM((128, 128), jnp.float32),
            pltpu.VMEM((128, 128), jnp.float32),
            pltpu.VMEM((CAP_C,), jnp.float32),
            pltpu.VMEM((CAP_C,), jnp.int32),
            pltpu.VMEM((OUT_PAD,), jnp.float32),
            pltpu.VMEM((OUT_PAD,), jnp.int32),
            pltpu.VMEM((OUT_PAD,), jnp.int32),
            pltpu.VMEM((OUT_PAD,), jnp.float32),
            pltpu.VMEM((384,), jnp.int32),
            pltpu.VMEM((OUT_PAD,), jnp.int32),
            pltpu.VMEM((384, 128), jnp.float32),
            pltpu.VMEM((OUT_PAD * 4,), jnp.float32),
            pltpu.SemaphoreType.DMA,
            pltpu.SemaphoreType.DMA,
        ],
    )(functools.partial(_sc_body, nb=nb, q=q, c=c, gp=gp))

    labels_p, boxes_p, scores_p = sc(octmax, thresh, lin, boxes128, scale16)
    return (labels_p[:, :NUM_TOP],
            boxes_p.reshape(nb, OUT_PAD, 4)[:, :NUM_TOP],
            scores_p[:, :NUM_TOP])
